# baseline (speedup 1.0000x reference)
import functools

import jax
import jax.numpy as jnp
from jax import lax
from jax.experimental import pallas as pl
from jax.experimental.pallas import tpu as pltpu
from jax.experimental.pallas import tpu_sc as plsc

N_NODES = 10000
N_EDGES = 320000
D_FEAT = 128

NC = 2
NS = 16
NW = NC * NS

CH = 128
NCHUNKS = -(-N_EDGES // CH)
NCHUNKS_PAD = -(-NCHUNKS // (2 * NW)) * (2 * NW)
CPW = NCHUNKS_PAD // NW

NPAD = 10240
SENTINEL = NPAD - 1
ROWS_PER_TILE = NPAD // NS
DRAIN_BLK = 128
N_DRAIN = ROWS_PER_TILE // DRAIN_BLK

_mesh = plsc.VectorSubcoreMesh(core_axis_name="c", subcore_axis_name="s",
                               num_cores=NC, num_subcores=NS)


@functools.partial(
    pl.kernel,
    out_type=(
        jax.ShapeDtypeStruct((NC, NPAD), jnp.float32),
        jax.ShapeDtypeStruct((NC, NPAD), jnp.float32),
    ),
    mesh=_mesh,
    scratch_types=[
        pltpu.VMEM((CPW, CH), jnp.int32),
        pltpu.VMEM((CPW, CH), jnp.int32),
        pltpu.VMEM((CH,), jnp.float32),
        pltpu.VMEM((ROWS_PER_TILE,), jnp.float32),
        pltpu.VMEM_SHARED((NPAD,), jnp.float32),
        pltpu.VMEM_SHARED((NPAD,), jnp.float32),
        pltpu.SemaphoreType.DMA,
    ],
)
def _deg_kernel(edge_hbm, degr_hbm, degc_hbm,
                ridx2, cidx2, ones_v, bounce, degr_sh, degc_sh, ssem):
    cid = lax.axis_index("c")
    sid = lax.axis_index("s")
    wid = cid * NS + sid

    def fill16(i, _):
        bounce[pl.ds(i * 16, 16)] = jnp.zeros((16,), jnp.float32)
        return 0
    lax.fori_loop(0, ROWS_PER_TILE // 16, fill16, 0)
    for i in range(CH // 16):
        ones_v[pl.ds(i * 16, 16)] = jnp.ones((16,), jnp.float32)

    tile_base = sid * ROWS_PER_TILE
    pltpu.sync_copy(bounce, degr_sh.at[pl.ds(tile_base, ROWS_PER_TILE)])
    pltpu.sync_copy(bounce, degc_sh.at[pl.ds(tile_base, ROWS_PER_TILE)])

    pltpu.sync_copy(edge_hbm.at[0, 2 * wid], ridx2.at[pl.ds(0, CPW // 2)])
    pltpu.sync_copy(edge_hbm.at[0, 2 * wid + 1], ridx2.at[pl.ds(CPW // 2, CPW // 2)])
    pltpu.sync_copy(edge_hbm.at[1, 2 * wid], cidx2.at[pl.ds(0, CPW // 2)])
    pltpu.sync_copy(edge_hbm.at[1, 2 * wid + 1], cidx2.at[pl.ds(CPW // 2, CPW // 2)])
    plsc.subcore_barrier()

    def body(j, _):
        pltpu.async_copy(ones_v, degr_sh.at[ridx2.at[j]], ssem, add=True)
        pltpu.async_copy(ones_v, degc_sh.at[cidx2.at[j]], ssem, add=True)

        @pl.when(j > 0)
        def _():
            pltpu.make_async_copy(ones_v, degr_sh.at[ridx2.at[j - 1]], ssem).wait()
            pltpu.make_async_copy(ones_v, degc_sh.at[cidx2.at[j - 1]], ssem).wait()
        return 0
    lax.fori_loop(0, CPW, body, 0)
    pltpu.make_async_copy(ones_v, degr_sh.at[ridx2.at[CPW - 1]], ssem).wait()
    pltpu.make_async_copy(ones_v, degc_sh.at[cidx2.at[CPW - 1]], ssem).wait()

    plsc.subcore_barrier()
    pltpu.sync_copy(degr_sh.at[pl.ds(tile_base, ROWS_PER_TILE)], bounce)
    pltpu.sync_copy(bounce, degr_hbm.at[cid, pl.ds(tile_base, ROWS_PER_TILE)])
    pltpu.sync_copy(degc_sh.at[pl.ds(tile_base, ROWS_PER_TILE)], bounce)
    pltpu.sync_copy(bounce, degc_hbm.at[cid, pl.ds(tile_base, ROWS_PER_TILE)])


@functools.partial(
    pl.kernel,
    out_type=(
        jax.ShapeDtypeStruct((NC, NPAD, D_FEAT), jnp.float32),
    ),
    mesh=_mesh,
    scratch_types=[
        pltpu.VMEM((CPW // 2, CH), jnp.int32),
        pltpu.VMEM((CPW // 2, CH), jnp.int32),
        pltpu.VMEM((2, CH, D_FEAT), jnp.float32),
        pltpu.VMEM_SHARED((NPAD, D_FEAT), jnp.float32),
        pltpu.SemaphoreType.DMA,
        pltpu.SemaphoreType.DMA,
        pltpu.SemaphoreType.DMA,
    ],
)
def _spmm_kernel(scaled_hbm, edge_hbm, out_hbm,
                 ridx2, cidx2, gbuf, acc_sh, gsem, ssem, dsem):
    cid = lax.axis_index("c")
    sid = lax.axis_index("s")
    wid = cid * NS + sid

    def fill16(r, _):
        for k in range(D_FEAT // 16):
            gbuf[0, r, pl.ds(k * 16, 16)] = jnp.zeros((16,), jnp.float32)
        return 0
    lax.fori_loop(0, CH, fill16, 0)

    tile_base = sid * ROWS_PER_TILE
    for k in range(N_DRAIN):
        pltpu.sync_copy(gbuf.at[0],
                        acc_sh.at[pl.ds(tile_base + k * DRAIN_BLK, DRAIN_BLK)])

    plsc.subcore_barrier()

    SCH = CPW // 2
    for h in range(2):
        pltpu.sync_copy(edge_hbm.at[0, 2 * wid + h], ridx2)
        pltpu.sync_copy(edge_hbm.at[1, 2 * wid + h], cidx2)
        pltpu.async_copy(scaled_hbm.at[cidx2.at[0]], gbuf.at[0], gsem)

        def body(j, _):
            b = j % 2

            @pl.when(j > 0)
            def _():
                pltpu.make_async_copy(gbuf.at[1 - b], acc_sh.at[ridx2.at[j - 1]],
                                      ssem).wait()

            @pl.when(j + 1 < SCH)
            def _():
                pltpu.async_copy(scaled_hbm.at[cidx2.at[j + 1]], gbuf.at[1 - b],
                                 gsem)

            pltpu.make_async_copy(scaled_hbm.at[cidx2.at[j]], gbuf.at[b],
                                  gsem).wait()
            pltpu.async_copy(gbuf.at[b], acc_sh.at[ridx2.at[j]], ssem, add=True)
            return 0
        lax.fori_loop(0, SCH, body, 0)
        pltpu.make_async_copy(gbuf.at[(SCH - 1) % 2], acc_sh.at[ridx2.at[SCH - 1]],
                              ssem).wait()

    plsc.subcore_barrier()
    for k in range(N_DRAIN):
        b = k % 2
        rows = pl.ds(tile_base + k * DRAIN_BLK, DRAIN_BLK)
        if k >= 2:
            prev = pl.ds(tile_base + (k - 2) * DRAIN_BLK, DRAIN_BLK)
            pltpu.make_async_copy(gbuf.at[b], out_hbm.at[cid, prev], dsem).wait()
        pltpu.sync_copy(acc_sh.at[rows], gbuf.at[b])
        pltpu.async_copy(gbuf.at[b], out_hbm.at[cid, rows], dsem)
    for k in range(N_DRAIN - 2, N_DRAIN):
        rows = pl.ds(tile_base + k * DRAIN_BLK, DRAIN_BLK)
        pltpu.make_async_copy(gbuf.at[k % 2], out_hbm.at[cid, rows], dsem).wait()


def _prep_body(degr_ref, degc_ref, emb_ref, scaled_ref, rrow_ref):
    degr = degr_ref[0] + degr_ref[1]
    degc = degc_ref[0] + degc_ref[1]
    rrow_ref[...] = lax.rsqrt(jnp.maximum(degr, 1.0))
    rcol = lax.rsqrt(jnp.maximum(degc, 1.0))
    rcol_n = rcol.reshape(NPAD)[:N_NODES]
    scaled_ref[:N_NODES, :] = emb_ref[...] * rcol_n[:, None]
    scaled_ref[N_NODES:, :] = jnp.zeros((NPAD - N_NODES, D_FEAT), jnp.float32)


def _finish_body(parts_ref, rrow_ref, out_ref):
    acc = parts_ref[0, :N_NODES, :] + parts_ref[1, :N_NODES, :]
    rrow = rrow_ref[...].reshape(NPAD)[:N_NODES]
    out_ref[...] = acc * rrow[:, None]


def kernel(embeddings, edge_index):
    pad = jnp.full((2, NCHUNKS_PAD * CH - N_EDGES), SENTINEL, jnp.int32)
    edge3 = jnp.concatenate([edge_index, pad], axis=1).reshape(
        2, NW * 2, CPW // 2, CH)
    degr_p, degc_p = _deg_kernel(edge3)
    scaled, rrow = pl.pallas_call(
        _prep_body,
        out_shape=(
            jax.ShapeDtypeStruct((NPAD, D_FEAT), jnp.float32),
            jax.ShapeDtypeStruct((NPAD // 128, 128), jnp.float32),
        ),
    )(degr_p.reshape(NC, NPAD // 128, 128),
      degc_p.reshape(NC, NPAD // 128, 128),
      embeddings)
    (parts,) = _spmm_kernel(scaled, edge3)
    out = pl.pallas_call(
        _finish_body,
        out_shape=jax.ShapeDtypeStruct((N_NODES, D_FEAT), jnp.float32),
    )(parts, rrow)
    return out


# speedup vs baseline: 1.0005x; 1.0005x over previous
import functools

import jax
import jax.numpy as jnp
from jax import lax
from jax.experimental import pallas as pl
from jax.experimental.pallas import tpu as pltpu
from jax.experimental.pallas import tpu_sc as plsc

N_NODES = 10000
N_EDGES = 320000
D_FEAT = 128

NC = 2
NS = 16
NW = NC * NS

CH = 128
NCHUNKS = -(-N_EDGES // CH)
NCHUNKS_PAD = -(-NCHUNKS // (2 * NW)) * (2 * NW)
CPW = NCHUNKS_PAD // NW

NPAD = 10240
SENTINEL = NPAD - 1
ROWS_PER_TILE = NPAD // NS
DRAIN_BLK = 128
N_DRAIN = ROWS_PER_TILE // DRAIN_BLK

_mesh = plsc.VectorSubcoreMesh(core_axis_name="c", subcore_axis_name="s",
                               num_cores=NC, num_subcores=NS)


@functools.partial(
    pl.kernel,
    out_type=(
        jax.ShapeDtypeStruct((NC, NPAD), jnp.float32),
        jax.ShapeDtypeStruct((NC, NPAD), jnp.float32),
    ),
    mesh=_mesh,
    scratch_types=[
        pltpu.VMEM((CPW, CH), jnp.int32),
        pltpu.VMEM((CPW, CH), jnp.int32),
        pltpu.VMEM((CH,), jnp.float32),
        pltpu.VMEM((ROWS_PER_TILE,), jnp.float32),
        pltpu.VMEM_SHARED((NPAD,), jnp.float32),
        pltpu.VMEM_SHARED((NPAD,), jnp.float32),
        pltpu.SemaphoreType.DMA,
    ],
)
def _deg_kernel(edge_hbm, degr_hbm, degc_hbm,
                ridx2, cidx2, ones_v, bounce, degr_sh, degc_sh, ssem):
    cid = lax.axis_index("c")
    sid = lax.axis_index("s")
    wid = cid * NS + sid

    def fill16(i, _):
        bounce[pl.ds(i * 16, 16)] = jnp.zeros((16,), jnp.float32)
        return 0
    lax.fori_loop(0, ROWS_PER_TILE // 16, fill16, 0)
    for i in range(CH // 16):
        ones_v[pl.ds(i * 16, 16)] = jnp.ones((16,), jnp.float32)

    tile_base = sid * ROWS_PER_TILE
    pltpu.sync_copy(bounce, degr_sh.at[pl.ds(tile_base, ROWS_PER_TILE)])
    pltpu.sync_copy(bounce, degc_sh.at[pl.ds(tile_base, ROWS_PER_TILE)])

    pltpu.sync_copy(edge_hbm.at[0, 2 * wid], ridx2.at[pl.ds(0, CPW // 2)])
    pltpu.sync_copy(edge_hbm.at[0, 2 * wid + 1], ridx2.at[pl.ds(CPW // 2, CPW // 2)])
    pltpu.sync_copy(edge_hbm.at[1, 2 * wid], cidx2.at[pl.ds(0, CPW // 2)])
    pltpu.sync_copy(edge_hbm.at[1, 2 * wid + 1], cidx2.at[pl.ds(CPW // 2, CPW // 2)])
    plsc.subcore_barrier()

    def body(j, _):
        pltpu.async_copy(ones_v, degr_sh.at[ridx2.at[j]], ssem, add=True)
        pltpu.async_copy(ones_v, degc_sh.at[cidx2.at[j]], ssem, add=True)

        @pl.when(j > 0)
        def _():
            pltpu.make_async_copy(ones_v, degr_sh.at[ridx2.at[j - 1]], ssem).wait()
            pltpu.make_async_copy(ones_v, degc_sh.at[cidx2.at[j - 1]], ssem).wait()
        return 0
    lax.fori_loop(0, CPW, body, 0)
    pltpu.make_async_copy(ones_v, degr_sh.at[ridx2.at[CPW - 1]], ssem).wait()
    pltpu.make_async_copy(ones_v, degc_sh.at[cidx2.at[CPW - 1]], ssem).wait()

    plsc.subcore_barrier()
    pltpu.sync_copy(degr_sh.at[pl.ds(tile_base, ROWS_PER_TILE)], bounce)
    pltpu.sync_copy(bounce, degr_hbm.at[cid, pl.ds(tile_base, ROWS_PER_TILE)])
    pltpu.sync_copy(degc_sh.at[pl.ds(tile_base, ROWS_PER_TILE)], bounce)
    pltpu.sync_copy(bounce, degc_hbm.at[cid, pl.ds(tile_base, ROWS_PER_TILE)])


@functools.partial(
    pl.kernel,
    out_type=(
        jax.ShapeDtypeStruct((NC, NPAD, D_FEAT), jnp.float32),
    ),
    mesh=_mesh,
    scratch_types=[
        pltpu.VMEM((CPW // 2, CH), jnp.int32),
        pltpu.VMEM((CPW // 2, CH), jnp.int32),
        pltpu.VMEM((2, CH, D_FEAT), jnp.float32),
        pltpu.VMEM_SHARED((NPAD, D_FEAT), jnp.float32),
        pltpu.SemaphoreType.DMA,
        pltpu.SemaphoreType.DMA,
        pltpu.SemaphoreType.DMA,
    ],
)
def _spmm_kernel(scaled_hbm, edge_hbm, out_hbm,
                 ridx2, cidx2, gbuf, acc_sh, gsem, ssem, dsem):
    cid = lax.axis_index("c")
    sid = lax.axis_index("s")
    wid = cid * NS + sid

    def fill16(r, _):
        for k in range(D_FEAT // 16):
            gbuf[0, r, pl.ds(k * 16, 16)] = jnp.zeros((16,), jnp.float32)
        return 0
    lax.fori_loop(0, CH, fill16, 0)

    tile_base = sid * ROWS_PER_TILE
    for k in range(N_DRAIN):
        pltpu.sync_copy(gbuf.at[0],
                        acc_sh.at[pl.ds(tile_base + k * DRAIN_BLK, DRAIN_BLK)])

    plsc.subcore_barrier()

    SCH = CPW // 2
    for h in range(2):
        pltpu.sync_copy(edge_hbm.at[0, 2 * wid + h], ridx2)
        pltpu.sync_copy(edge_hbm.at[1, 2 * wid + h], cidx2)
        pltpu.async_copy(scaled_hbm.at[cidx2.at[0]], gbuf.at[0], gsem)

        def body(j, _):
            b = j % 2

            @pl.when(j + 1 < SCH)
            def _():
                pltpu.async_copy(scaled_hbm.at[cidx2.at[j + 1]], gbuf.at[1 - b],
                                 gsem)

            pltpu.make_async_copy(scaled_hbm.at[cidx2.at[j]], gbuf.at[b],
                                  gsem).wait()
            pltpu.sync_copy(gbuf.at[b], acc_sh.at[ridx2.at[j]], add=True)
            return 0
        lax.fori_loop(0, SCH, body, 0)

    plsc.subcore_barrier()
    for k in range(N_DRAIN):
        b = k % 2
        rows = pl.ds(tile_base + k * DRAIN_BLK, DRAIN_BLK)
        if k >= 2:
            prev = pl.ds(tile_base + (k - 2) * DRAIN_BLK, DRAIN_BLK)
            pltpu.make_async_copy(gbuf.at[b], out_hbm.at[cid, prev], dsem).wait()
        pltpu.sync_copy(acc_sh.at[rows], gbuf.at[b])
        pltpu.async_copy(gbuf.at[b], out_hbm.at[cid, rows], dsem)
    for k in range(N_DRAIN - 2, N_DRAIN):
        rows = pl.ds(tile_base + k * DRAIN_BLK, DRAIN_BLK)
        pltpu.make_async_copy(gbuf.at[k % 2], out_hbm.at[cid, rows], dsem).wait()


def _prep_body(degr_ref, degc_ref, emb_ref, scaled_ref, rrow_ref):
    degr = degr_ref[0] + degr_ref[1]
    degc = degc_ref[0] + degc_ref[1]
    rrow_ref[...] = lax.rsqrt(jnp.maximum(degr, 1.0))
    rcol = lax.rsqrt(jnp.maximum(degc, 1.0))
    rcol_n = rcol.reshape(NPAD)[:N_NODES]
    scaled_ref[:N_NODES, :] = emb_ref[...] * rcol_n[:, None]
    scaled_ref[N_NODES:, :] = jnp.zeros((NPAD - N_NODES, D_FEAT), jnp.float32)


def _finish_body(parts_ref, rrow_ref, out_ref):
    acc = parts_ref[0, :N_NODES, :] + parts_ref[1, :N_NODES, :]
    rrow = rrow_ref[...].reshape(NPAD)[:N_NODES]
    out_ref[...] = acc * rrow[:, None]


def kernel(embeddings, edge_index):
    pad = jnp.full((2, NCHUNKS_PAD * CH - N_EDGES), SENTINEL, jnp.int32)
    edge3 = jnp.concatenate([edge_index, pad], axis=1).reshape(
        2, NW * 2, CPW // 2, CH)
    degr_p, degc_p = _deg_kernel(edge3)
    scaled, rrow = pl.pallas_call(
        _prep_body,
        out_shape=(
            jax.ShapeDtypeStruct((NPAD, D_FEAT), jnp.float32),
            jax.ShapeDtypeStruct((NPAD // 128, 128), jnp.float32),
        ),
    )(degr_p.reshape(NC, NPAD // 128, 128),
      degc_p.reshape(NC, NPAD // 128, 128),
      embeddings)
    (parts,) = _spmm_kernel(scaled, edge3)
    out = pl.pallas_call(
        _finish_body,
        out_shape=jax.ShapeDtypeStruct((N_NODES, D_FEAT), jnp.float32),
    )(parts, rrow)
    return out


# speedup vs baseline: 1.5201x; 1.5194x over previous
import functools

import jax
import jax.numpy as jnp
from jax import lax
from jax.experimental import pallas as pl
from jax.experimental.pallas import tpu as pltpu
from jax.experimental.pallas import tpu_sc as plsc

N_NODES = 10000
N_EDGES = 320000
D_FEAT = 128

NC = 2
NS = 16
NW = NC * NS

CH = 128
NCHUNKS = N_EDGES // CH
CHUNKS_Q, CHUNKS_R = divmod(NCHUNKS, NW)

NPAD = 10240
ROWS_PER_TILE = NPAD // NS
DRAIN_BLK = 128
N_DRAIN = ROWS_PER_TILE // DRAIN_BLK

_mesh = plsc.VectorSubcoreMesh(core_axis_name="c", subcore_axis_name="s",
                               num_cores=NC, num_subcores=NS)


def _worker_chunks(wid):
    start = wid * CHUNKS_Q + jnp.minimum(wid, CHUNKS_R)
    count = CHUNKS_Q + (wid < CHUNKS_R).astype(jnp.int32)
    return start, count


@functools.partial(
    pl.kernel,
    out_type=(
        jax.ShapeDtypeStruct((NC, NPAD), jnp.float32),
        jax.ShapeDtypeStruct((NC, NPAD), jnp.float32),
    ),
    mesh=_mesh,
    scratch_types=[
        pltpu.VMEM((1, CH), jnp.int32),
        pltpu.VMEM((1, CH), jnp.int32),
        pltpu.VMEM((CH,), jnp.float32),
        pltpu.VMEM((ROWS_PER_TILE,), jnp.float32),
        pltpu.VMEM_SHARED((NPAD,), jnp.float32),
        pltpu.VMEM_SHARED((NPAD,), jnp.float32),
    ],
)
def _deg_kernel(edge_hbm, degr_hbm, degc_hbm,
                ridx, cidx, ones_v, bounce, degr_sh, degc_sh):
    cid = lax.axis_index("c")
    sid = lax.axis_index("s")
    wid = cid * NS + sid

    def fill16(i, _):
        bounce[pl.ds(i * 16, 16)] = jnp.zeros((16,), jnp.float32)
        return 0
    lax.fori_loop(0, ROWS_PER_TILE // 16, fill16, 0)
    for i in range(CH // 16):
        ones_v[pl.ds(i * 16, 16)] = jnp.ones((16,), jnp.float32)

    tile_base = sid * ROWS_PER_TILE
    pltpu.sync_copy(bounce, degr_sh.at[pl.ds(tile_base, ROWS_PER_TILE)])
    pltpu.sync_copy(bounce, degc_sh.at[pl.ds(tile_base, ROWS_PER_TILE)])
    plsc.subcore_barrier()

    start, count = _worker_chunks(wid)

    def body(j, _):
        base = (start + j) * CH
        pltpu.sync_copy(edge_hbm.at[0, pl.ds(base, CH)], ridx.at[0])
        pltpu.sync_copy(edge_hbm.at[1, pl.ds(base, CH)], cidx.at[0])
        pltpu.sync_copy(ones_v, degr_sh.at[ridx.at[0]], add=True)
        pltpu.sync_copy(ones_v, degc_sh.at[cidx.at[0]], add=True)
        return 0
    lax.fori_loop(0, count, body, 0)

    plsc.subcore_barrier()
    pltpu.sync_copy(degr_sh.at[pl.ds(tile_base, ROWS_PER_TILE)], bounce)
    pltpu.sync_copy(bounce, degr_hbm.at[cid, pl.ds(tile_base, ROWS_PER_TILE)])
    pltpu.sync_copy(degc_sh.at[pl.ds(tile_base, ROWS_PER_TILE)], bounce)
    pltpu.sync_copy(bounce, degc_hbm.at[cid, pl.ds(tile_base, ROWS_PER_TILE)])


@functools.partial(
    pl.kernel,
    out_type=(
        jax.ShapeDtypeStruct((NC, NPAD, D_FEAT), jnp.float32),
    ),
    mesh=_mesh,
    scratch_types=[
        pltpu.VMEM((1, CH), jnp.int32),
        pltpu.VMEM((1, CH), jnp.int32),
        pltpu.VMEM((CH, D_FEAT), jnp.float32),
        pltpu.VMEM_SHARED((NPAD, D_FEAT), jnp.float32),
        pltpu.SemaphoreType.DMA,
    ],
)
def _spmm_kernel(scaled_hbm, edge_hbm, out_hbm,
                 ridx, cidx, gbuf, acc_sh, gsem):
    cid = lax.axis_index("c")
    sid = lax.axis_index("s")
    wid = cid * NS + sid

    def fill16(i, _):
        r = i // (D_FEAT // 16)
        k = i % (D_FEAT // 16)
        gbuf[r, pl.ds(k * 16, 16)] = jnp.zeros((16,), jnp.float32)
        return 0
    lax.fori_loop(0, DRAIN_BLK * (D_FEAT // 16), fill16, 0)

    tile_base = sid * ROWS_PER_TILE
    for k in range(N_DRAIN):
        pltpu.sync_copy(gbuf, acc_sh.at[pl.ds(tile_base + k * DRAIN_BLK, DRAIN_BLK)])
    plsc.subcore_barrier()

    start, count = _worker_chunks(wid)

    def body(j, _):
        base = (start + j) * CH
        pltpu.sync_copy(edge_hbm.at[1, pl.ds(base, CH)], cidx.at[0])
        pltpu.async_copy(scaled_hbm.at[cidx.at[0]], gbuf, gsem).wait()
        pltpu.sync_copy(edge_hbm.at[0, pl.ds(base, CH)], ridx.at[0])
        pltpu.sync_copy(gbuf, acc_sh.at[ridx.at[0]], add=True)
        return 0
    lax.fori_loop(0, count, body, 0)

    plsc.subcore_barrier()
    for k in range(N_DRAIN):
        rows = pl.ds(tile_base + k * DRAIN_BLK, DRAIN_BLK)
        pltpu.sync_copy(acc_sh.at[rows], gbuf)
        pltpu.sync_copy(gbuf, out_hbm.at[cid, rows])


def _prep_body(degr_ref, degc_ref, emb_ref, scaled_ref, rrow_ref):
    degr = degr_ref[0] + degr_ref[1]
    degc = degc_ref[0] + degc_ref[1]
    rrow_ref[...] = lax.rsqrt(jnp.maximum(degr, 1.0))
    rcol = lax.rsqrt(jnp.maximum(degc, 1.0))
    rcol_n = rcol.reshape(NPAD)[:N_NODES]
    scaled_ref[...] = emb_ref[...] * rcol_n[:, None]


def _finish_body(parts_ref, rrow_ref, out_ref):
    acc = parts_ref[0, :N_NODES, :] + parts_ref[1, :N_NODES, :]
    rrow = rrow_ref[...].reshape(NPAD)[:N_NODES]
    out_ref[...] = acc * rrow[:, None]


def kernel(embeddings, edge_index):
    degr_p, degc_p = _deg_kernel(edge_index)
    scaled, rrow = pl.pallas_call(
        _prep_body,
        out_shape=(
            jax.ShapeDtypeStruct((N_NODES, D_FEAT), jnp.float32),
            jax.ShapeDtypeStruct((NPAD // 128, 128), jnp.float32),
        ),
    )(degr_p.reshape(NC, NPAD // 128, 128),
      degc_p.reshape(NC, NPAD // 128, 128),
      embeddings)
    (parts,) = _spmm_kernel(scaled, edge_index)
    out = pl.pallas_call(
        _finish_body,
        out_shape=jax.ShapeDtypeStruct((N_NODES, D_FEAT), jnp.float32),
    )(parts, rrow)
    return out


# speedup vs baseline: 3.5536x; 2.3377x over previous
import functools

import jax
import jax.numpy as jnp
from jax import lax
from jax.experimental import pallas as pl
from jax.experimental.pallas import tpu as pltpu
from jax.experimental.pallas import tpu_sc as plsc

N_NODES = 10000
N_EDGES = 320000
D_FEAT = 128

NC = 2
NS = 16
NW = NC * NS

CH = 128
NCHUNKS = -(-N_EDGES // CH)
NCHUNKS_PAD = -(-NCHUNKS // (2 * NW)) * (2 * NW)
CPW = NCHUNKS_PAD // NW

NPAD = 10240
ROWS_PER_TILE = NPAD // NS
DRAIN_BLK = 128
N_DRAIN = ROWS_PER_TILE // DRAIN_BLK

_mesh = plsc.VectorSubcoreMesh(core_axis_name="c", subcore_axis_name="s",
                               num_cores=NC, num_subcores=NS)


@functools.partial(
    pl.kernel,
    out_type=(
        jax.ShapeDtypeStruct((NC, NPAD), jnp.float32),
        jax.ShapeDtypeStruct((NC, NPAD), jnp.float32),
    ),
    mesh=_mesh,
    scratch_types=[
        pltpu.VMEM((CPW, CH), jnp.int32),
        pltpu.VMEM((CPW, CH), jnp.int32),
        pltpu.VMEM((CH,), jnp.float32),
        pltpu.VMEM((ROWS_PER_TILE,), jnp.float32),
        pltpu.VMEM_SHARED((NPAD,), jnp.float32),
        pltpu.VMEM_SHARED((NPAD,), jnp.float32),
        pltpu.SemaphoreType.DMA,
    ],
)
def _deg_kernel(edge_hbm, degr_hbm, degc_hbm,
                ridx2, cidx2, ones_v, bounce, degr_sh, degc_sh, ssem):
    cid = lax.axis_index("c")
    sid = lax.axis_index("s")
    wid = cid * NS + sid

    def fill16(i, _):
        bounce[pl.ds(i * 16, 16)] = jnp.zeros((16,), jnp.float32)
        return 0
    lax.fori_loop(0, ROWS_PER_TILE // 16, fill16, 0)
    for i in range(CH // 16):
        ones_v[pl.ds(i * 16, 16)] = jnp.ones((16,), jnp.float32)

    tile_base = sid * ROWS_PER_TILE
    pltpu.sync_copy(bounce, degr_sh.at[pl.ds(tile_base, ROWS_PER_TILE)])
    pltpu.sync_copy(bounce, degc_sh.at[pl.ds(tile_base, ROWS_PER_TILE)])

    pltpu.sync_copy(edge_hbm.at[0, 2 * wid], ridx2.at[pl.ds(0, CPW // 2)])
    pltpu.sync_copy(edge_hbm.at[0, 2 * wid + 1], ridx2.at[pl.ds(CPW // 2, CPW // 2)])
    pltpu.sync_copy(edge_hbm.at[1, 2 * wid], cidx2.at[pl.ds(0, CPW // 2)])
    pltpu.sync_copy(edge_hbm.at[1, 2 * wid + 1], cidx2.at[pl.ds(CPW // 2, CPW // 2)])
    plsc.subcore_barrier()

    def body(j, _):
        pltpu.async_copy(ones_v, degr_sh.at[ridx2.at[j]], ssem, add=True)
        pltpu.async_copy(ones_v, degc_sh.at[cidx2.at[j]], ssem, add=True)

        @pl.when(j > 0)
        def _():
            pltpu.make_async_copy(ones_v, degr_sh.at[ridx2.at[j - 1]], ssem).wait()
            pltpu.make_async_copy(ones_v, degc_sh.at[cidx2.at[j - 1]], ssem).wait()
        return 0
    lax.fori_loop(0, CPW, body, 0)
    pltpu.make_async_copy(ones_v, degr_sh.at[ridx2.at[CPW - 1]], ssem).wait()
    pltpu.make_async_copy(ones_v, degc_sh.at[cidx2.at[CPW - 1]], ssem).wait()

    plsc.subcore_barrier()
    pltpu.sync_copy(degr_sh.at[pl.ds(tile_base, ROWS_PER_TILE)], bounce)
    pltpu.sync_copy(bounce, degr_hbm.at[cid, pl.ds(tile_base, ROWS_PER_TILE)])
    pltpu.sync_copy(degc_sh.at[pl.ds(tile_base, ROWS_PER_TILE)], bounce)
    pltpu.sync_copy(bounce, degc_hbm.at[cid, pl.ds(tile_base, ROWS_PER_TILE)])


@functools.partial(
    pl.kernel,
    out_type=(
        jax.ShapeDtypeStruct((NC, NPAD, D_FEAT), jnp.float32),
    ),
    mesh=_mesh,
    scratch_types=[
        pltpu.VMEM((CPW // 2, CH), jnp.int32),
        pltpu.VMEM((CPW // 2, CH), jnp.int32),
        pltpu.VMEM((2, CH, D_FEAT), jnp.float32),
        pltpu.VMEM_SHARED((NPAD, D_FEAT), jnp.float32),
        pltpu.SemaphoreType.DMA,
        pltpu.SemaphoreType.DMA,
        pltpu.SemaphoreType.DMA,
    ],
)
def _spmm_kernel(scaled_hbm, edge_hbm, out_hbm,
                 ridx2, cidx2, gbuf, acc_sh, gsem, ssem, dsem):
    cid = lax.axis_index("c")
    sid = lax.axis_index("s")
    wid = cid * NS + sid

    def fill16(r, _):
        for k in range(D_FEAT // 16):
            gbuf[0, r, pl.ds(k * 16, 16)] = jnp.zeros((16,), jnp.float32)
        return 0
    lax.fori_loop(0, CH, fill16, 0)

    tile_base = sid * ROWS_PER_TILE
    for k in range(N_DRAIN):
        pltpu.sync_copy(gbuf.at[0],
                        acc_sh.at[pl.ds(tile_base + k * DRAIN_BLK, DRAIN_BLK)])

    plsc.subcore_barrier()

    SCH = CPW // 2
    for h in range(2):
        pltpu.sync_copy(edge_hbm.at[0, 2 * wid + h], ridx2)
        pltpu.sync_copy(edge_hbm.at[1, 2 * wid + h], cidx2)
        pltpu.async_copy(scaled_hbm.at[cidx2.at[0]], gbuf.at[0], gsem)

        def body(j, _):
            b = j % 2

            @pl.when(j + 1 < SCH)
            def _():
                pltpu.async_copy(scaled_hbm.at[cidx2.at[j + 1]], gbuf.at[1 - b],
                                 gsem)

            pltpu.make_async_copy(scaled_hbm.at[cidx2.at[j]], gbuf.at[b],
                                  gsem).wait()
            pltpu.sync_copy(gbuf.at[b], acc_sh.at[ridx2.at[j]], add=True)
            return 0
        lax.fori_loop(0, SCH, body, 0)

    plsc.subcore_barrier()
    for k in range(N_DRAIN):
        b = k % 2
        rows = pl.ds(tile_base + k * DRAIN_BLK, DRAIN_BLK)
        if k >= 2:
            prev = pl.ds(tile_base + (k - 2) * DRAIN_BLK, DRAIN_BLK)
            pltpu.make_async_copy(gbuf.at[b], out_hbm.at[cid, prev], dsem).wait()
        pltpu.sync_copy(acc_sh.at[rows], gbuf.at[b])
        pltpu.async_copy(gbuf.at[b], out_hbm.at[cid, rows], dsem)
    for k in range(N_DRAIN - 2, N_DRAIN):
        rows = pl.ds(tile_base + k * DRAIN_BLK, DRAIN_BLK)
        pltpu.make_async_copy(gbuf.at[k % 2], out_hbm.at[cid, rows], dsem).wait()


def _prep_body(degr_ref, degc_ref, emb_ref, scaled_ref, rrow_ref):
    degr = degr_ref[0] + degr_ref[1]
    degc = degc_ref[0] + degc_ref[1]
    rrow_ref[...] = lax.rsqrt(jnp.maximum(degr, 1.0))
    rcol = lax.rsqrt(jnp.maximum(degc, 1.0))
    rcol_n = rcol.reshape(NPAD)[:N_NODES]
    scaled_ref[:N_NODES, :] = emb_ref[...] * rcol_n[:, None]
    scaled_ref[N_NODES:, :] = jnp.zeros((NPAD - N_NODES, D_FEAT), jnp.float32)


def _finish_body(parts_ref, rrow_ref, out_ref):
    acc = parts_ref[0, :N_NODES, :] + parts_ref[1, :N_NODES, :]
    rrow = rrow_ref[...].reshape(NPAD)[:N_NODES]
    out_ref[...] = acc * rrow[:, None]


def kernel(embeddings, edge_index):
    npad_e = NCHUNKS_PAD * CH - N_EDGES
    pad_row = N_NODES + jnp.arange(npad_e, dtype=jnp.int32) % (NPAD - N_NODES)
    pad = jnp.broadcast_to(pad_row, (2, npad_e))
    edge3 = jnp.concatenate([edge_index, pad], axis=1).reshape(
        2, NW * 2, CPW // 2, CH)
    degr_p, degc_p = _deg_kernel(edge3)
    scaled, rrow = pl.pallas_call(
        _prep_body,
        out_shape=(
            jax.ShapeDtypeStruct((NPAD, D_FEAT), jnp.float32),
            jax.ShapeDtypeStruct((NPAD // 128, 128), jnp.float32),
        ),
    )(degr_p.reshape(NC, NPAD // 128, 128),
      degc_p.reshape(NC, NPAD // 128, 128),
      embeddings)
    (parts,) = _spmm_kernel(scaled, edge3)
    out = pl.pallas_call(
        _finish_body,
        out_shape=jax.ShapeDtypeStruct((N_NODES, D_FEAT), jnp.float32),
    )(parts, rrow)
    return out


# speedup vs baseline: 3.5603x; 1.0019x over previous
import functools

import jax
import jax.numpy as jnp
from jax import lax
from jax.experimental import pallas as pl
from jax.experimental.pallas import tpu as pltpu
from jax.experimental.pallas import tpu_sc as plsc

N_NODES = 10000
N_EDGES = 320000
D_FEAT = 128

NC = 2
NS = 16
NW = NC * NS

CH = 128
NCHUNKS = -(-N_EDGES // CH)
NCHUNKS_PAD = -(-NCHUNKS // (2 * NW)) * (2 * NW)
CPW = NCHUNKS_PAD // NW

NPAD = 10240
ROWS_PER_TILE = NPAD // NS
DRAIN_BLK = 128
N_DRAIN = ROWS_PER_TILE // DRAIN_BLK

_mesh = plsc.VectorSubcoreMesh(core_axis_name="c", subcore_axis_name="s",
                               num_cores=NC, num_subcores=NS)


@functools.partial(
    pl.kernel,
    out_type=(
        jax.ShapeDtypeStruct((NC, NPAD), jnp.float32),
        jax.ShapeDtypeStruct((NC, NPAD), jnp.float32),
    ),
    mesh=_mesh,
    scratch_types=[
        pltpu.VMEM((CPW, CH), jnp.int32),
        pltpu.VMEM((CPW, CH), jnp.int32),
        pltpu.VMEM((CH,), jnp.float32),
        pltpu.VMEM((ROWS_PER_TILE,), jnp.float32),
        pltpu.VMEM_SHARED((NPAD,), jnp.float32),
        pltpu.VMEM_SHARED((NPAD,), jnp.float32),
        pltpu.SemaphoreType.DMA,
    ],
)
def _deg_kernel(edge_hbm, degr_hbm, degc_hbm,
                ridx2, cidx2, ones_v, bounce, degr_sh, degc_sh, ssem):
    cid = lax.axis_index("c")
    sid = lax.axis_index("s")
    wid = cid * NS + sid

    def fill16(i, _):
        bounce[pl.ds(i * 16, 16)] = jnp.zeros((16,), jnp.float32)
        return 0
    lax.fori_loop(0, ROWS_PER_TILE // 16, fill16, 0)
    for i in range(CH // 16):
        ones_v[pl.ds(i * 16, 16)] = jnp.ones((16,), jnp.float32)

    tile_base = sid * ROWS_PER_TILE
    pltpu.sync_copy(bounce, degr_sh.at[pl.ds(tile_base, ROWS_PER_TILE)])
    pltpu.sync_copy(bounce, degc_sh.at[pl.ds(tile_base, ROWS_PER_TILE)])

    pltpu.sync_copy(edge_hbm.at[0, 2 * wid], ridx2.at[pl.ds(0, CPW // 2)])
    pltpu.sync_copy(edge_hbm.at[0, 2 * wid + 1], ridx2.at[pl.ds(CPW // 2, CPW // 2)])
    pltpu.sync_copy(edge_hbm.at[1, 2 * wid], cidx2.at[pl.ds(0, CPW // 2)])
    pltpu.sync_copy(edge_hbm.at[1, 2 * wid + 1], cidx2.at[pl.ds(CPW // 2, CPW // 2)])
    plsc.subcore_barrier()

    def body(j, _):
        pltpu.async_copy(ones_v, degr_sh.at[ridx2.at[j]], ssem, add=True)
        pltpu.async_copy(ones_v, degc_sh.at[cidx2.at[j]], ssem, add=True)

        @pl.when(j > 0)
        def _():
            pltpu.make_async_copy(ones_v, degr_sh.at[ridx2.at[j - 1]], ssem).wait()
            pltpu.make_async_copy(ones_v, degc_sh.at[cidx2.at[j - 1]], ssem).wait()
        return 0
    lax.fori_loop(0, CPW, body, 0)
    pltpu.make_async_copy(ones_v, degr_sh.at[ridx2.at[CPW - 1]], ssem).wait()
    pltpu.make_async_copy(ones_v, degc_sh.at[cidx2.at[CPW - 1]], ssem).wait()

    plsc.subcore_barrier()
    pltpu.sync_copy(degr_sh.at[pl.ds(tile_base, ROWS_PER_TILE)], bounce)
    pltpu.sync_copy(bounce, degr_hbm.at[cid, pl.ds(tile_base, ROWS_PER_TILE)])
    pltpu.sync_copy(degc_sh.at[pl.ds(tile_base, ROWS_PER_TILE)], bounce)
    pltpu.sync_copy(bounce, degc_hbm.at[cid, pl.ds(tile_base, ROWS_PER_TILE)])


@functools.partial(
    pl.kernel,
    out_type=(
        jax.ShapeDtypeStruct((NC, NPAD, D_FEAT), jnp.float32),
    ),
    mesh=_mesh,
    scratch_types=[
        pltpu.VMEM((CPW // 2, CH), jnp.int32),
        pltpu.VMEM((CPW // 2, CH), jnp.int32),
        pltpu.VMEM((2, CH, D_FEAT), jnp.float32),
        pltpu.VMEM_SHARED((NPAD, D_FEAT), jnp.float32),
        pltpu.SemaphoreType.DMA,
        pltpu.SemaphoreType.DMA,
        pltpu.SemaphoreType.DMA,
    ],
)
def _spmm_kernel(scaled_hbm, edge_hbm, out_hbm,
                 ridx2, cidx2, gbuf, acc_sh, gsem, ssem, dsem):
    cid = lax.axis_index("c")
    sid = lax.axis_index("s")
    wid = cid * NS + sid

    def fill16(r, _):
        for k in range(D_FEAT // 16):
            gbuf[0, r, pl.ds(k * 16, 16)] = jnp.zeros((16,), jnp.float32)
        return 0
    lax.fori_loop(0, CH, fill16, 0)

    tile_base = sid * ROWS_PER_TILE
    for k in range(N_DRAIN):
        pltpu.sync_copy(gbuf.at[0],
                        acc_sh.at[pl.ds(tile_base + k * DRAIN_BLK, DRAIN_BLK)])

    plsc.subcore_barrier()

    SCH = CPW // 2
    for h in range(2):
        pltpu.sync_copy(edge_hbm.at[0, 2 * wid + h], ridx2)
        pltpu.sync_copy(edge_hbm.at[1, 2 * wid + h], cidx2)
        pltpu.async_copy(scaled_hbm.at[cidx2.at[0]], gbuf.at[0], gsem)

        def body(j, _):
            b = j % 2

            @pl.when(j > 0)
            def _():
                pltpu.make_async_copy(gbuf.at[1 - b], acc_sh.at[ridx2.at[j - 1]],
                                      ssem).wait()

            @pl.when(j + 1 < SCH)
            def _():
                pltpu.async_copy(scaled_hbm.at[cidx2.at[j + 1]], gbuf.at[1 - b],
                                 gsem)

            pltpu.make_async_copy(scaled_hbm.at[cidx2.at[j]], gbuf.at[b],
                                  gsem).wait()
            pltpu.async_copy(gbuf.at[b], acc_sh.at[ridx2.at[j]], ssem, add=True)
            return 0
        lax.fori_loop(0, SCH, body, 0)
        pltpu.make_async_copy(gbuf.at[(SCH - 1) % 2],
                              acc_sh.at[ridx2.at[SCH - 1]], ssem).wait()

    plsc.subcore_barrier()
    for k in range(N_DRAIN):
        b = k % 2
        rows = pl.ds(tile_base + k * DRAIN_BLK, DRAIN_BLK)
        if k >= 2:
            prev = pl.ds(tile_base + (k - 2) * DRAIN_BLK, DRAIN_BLK)
            pltpu.make_async_copy(gbuf.at[b], out_hbm.at[cid, prev], dsem).wait()
        pltpu.sync_copy(acc_sh.at[rows], gbuf.at[b])
        pltpu.async_copy(gbuf.at[b], out_hbm.at[cid, rows], dsem)
    for k in range(N_DRAIN - 2, N_DRAIN):
        rows = pl.ds(tile_base + k * DRAIN_BLK, DRAIN_BLK)
        pltpu.make_async_copy(gbuf.at[k % 2], out_hbm.at[cid, rows], dsem).wait()


def _prep_body(degr_ref, degc_ref, emb_ref, scaled_ref, rrow_ref):
    degr = degr_ref[0] + degr_ref[1]
    degc = degc_ref[0] + degc_ref[1]
    rrow_ref[...] = lax.rsqrt(jnp.maximum(degr, 1.0))
    rcol = lax.rsqrt(jnp.maximum(degc, 1.0))
    rcol_n = rcol.reshape(NPAD)[:N_NODES]
    scaled_ref[:N_NODES, :] = emb_ref[...] * rcol_n[:, None]
    scaled_ref[N_NODES:, :] = jnp.zeros((NPAD - N_NODES, D_FEAT), jnp.float32)


def _finish_body(parts_ref, rrow_ref, out_ref):
    acc = parts_ref[0, :N_NODES, :] + parts_ref[1, :N_NODES, :]
    rrow = rrow_ref[...].reshape(NPAD)[:N_NODES]
    out_ref[...] = acc * rrow[:, None]


def kernel(embeddings, edge_index):
    npad_e = NCHUNKS_PAD * CH - N_EDGES
    pad_row = N_NODES + jnp.arange(npad_e, dtype=jnp.int32) % (NPAD - N_NODES)
    pad = jnp.broadcast_to(pad_row, (2, npad_e))
    edge3 = jnp.concatenate([edge_index, pad], axis=1).reshape(
        2, NW * 2, CPW // 2, CH)
    degr_p, degc_p = _deg_kernel(edge3)
    scaled, rrow = pl.pallas_call(
        _prep_body,
        out_shape=(
            jax.ShapeDtypeStruct((NPAD, D_FEAT), jnp.float32),
            jax.ShapeDtypeStruct((NPAD // 128, 128), jnp.float32),
        ),
    )(degr_p.reshape(NC, NPAD // 128, 128),
      degc_p.reshape(NC, NPAD // 128, 128),
      embeddings)
    (parts,) = _spmm_kernel(scaled, edge3)
    out = pl.pallas_call(
        _finish_body,
        out_shape=jax.ShapeDtypeStruct((N_NODES, D_FEAT), jnp.float32),
    )(parts, rrow)
    return out


# speedup vs baseline: 3.5680x; 1.0022x over previous
import functools

import jax
import jax.numpy as jnp
from jax import lax
from jax.experimental import pallas as pl
from jax.experimental.pallas import tpu as pltpu
from jax.experimental.pallas import tpu_sc as plsc

N_NODES = 10000
N_EDGES = 320000
D_FEAT = 128

NC = 2
NS = 16
NW = NC * NS

CH = 128
NCHUNKS = -(-N_EDGES // CH)
NCHUNKS_PAD = -(-NCHUNKS // (2 * NW)) * (2 * NW)
CPW = NCHUNKS_PAD // NW

NPAD = 10240
ROWS_PER_TILE = NPAD // NS
DRAIN_BLK = 128
N_DRAIN = ROWS_PER_TILE // DRAIN_BLK

_mesh = plsc.VectorSubcoreMesh(core_axis_name="c", subcore_axis_name="s",
                               num_cores=NC, num_subcores=NS)


@functools.partial(
    pl.kernel,
    out_type=(
        jax.ShapeDtypeStruct((NC, NPAD), jnp.float32),
        jax.ShapeDtypeStruct((NC, NPAD), jnp.float32),
    ),
    mesh=_mesh,
    scratch_types=[
        pltpu.VMEM((CPW, CH), jnp.int32),
        pltpu.VMEM((CPW, CH), jnp.int32),
        pltpu.VMEM((CH,), jnp.float32),
        pltpu.VMEM((ROWS_PER_TILE,), jnp.float32),
        pltpu.VMEM_SHARED((NPAD,), jnp.float32),
        pltpu.VMEM_SHARED((NPAD,), jnp.float32),
        pltpu.SemaphoreType.DMA,
    ],
)
def _deg_kernel(edge_hbm, degr_hbm, degc_hbm,
                ridx2, cidx2, ones_v, bounce, degr_sh, degc_sh, ssem):
    cid = lax.axis_index("c")
    sid = lax.axis_index("s")
    wid = cid * NS + sid

    def fill16(i, _):
        bounce[pl.ds(i * 16, 16)] = jnp.zeros((16,), jnp.float32)
        return 0
    lax.fori_loop(0, ROWS_PER_TILE // 16, fill16, 0)
    for i in range(CH // 16):
        ones_v[pl.ds(i * 16, 16)] = jnp.ones((16,), jnp.float32)

    tile_base = sid * ROWS_PER_TILE
    pltpu.sync_copy(bounce, degr_sh.at[pl.ds(tile_base, ROWS_PER_TILE)])
    pltpu.sync_copy(bounce, degc_sh.at[pl.ds(tile_base, ROWS_PER_TILE)])

    pltpu.sync_copy(edge_hbm.at[0, 2 * wid], ridx2.at[pl.ds(0, CPW // 2)])
    pltpu.sync_copy(edge_hbm.at[0, 2 * wid + 1], ridx2.at[pl.ds(CPW // 2, CPW // 2)])
    pltpu.sync_copy(edge_hbm.at[1, 2 * wid], cidx2.at[pl.ds(0, CPW // 2)])
    pltpu.sync_copy(edge_hbm.at[1, 2 * wid + 1], cidx2.at[pl.ds(CPW // 2, CPW // 2)])
    plsc.subcore_barrier()

    def body(j, _):
        pltpu.async_copy(ones_v, degr_sh.at[ridx2.at[j]], ssem, add=True)
        pltpu.async_copy(ones_v, degc_sh.at[cidx2.at[j]], ssem, add=True)

        @pl.when(j > 0)
        def _():
            pltpu.make_async_copy(ones_v, degr_sh.at[ridx2.at[j - 1]], ssem).wait()
            pltpu.make_async_copy(ones_v, degc_sh.at[cidx2.at[j - 1]], ssem).wait()
        return 0
    lax.fori_loop(0, CPW, body, 0)
    pltpu.make_async_copy(ones_v, degr_sh.at[ridx2.at[CPW - 1]], ssem).wait()
    pltpu.make_async_copy(ones_v, degc_sh.at[cidx2.at[CPW - 1]], ssem).wait()

    plsc.subcore_barrier()
    pltpu.sync_copy(degr_sh.at[pl.ds(tile_base, ROWS_PER_TILE)], bounce)
    pltpu.sync_copy(bounce, degr_hbm.at[cid, pl.ds(tile_base, ROWS_PER_TILE)])
    pltpu.sync_copy(degc_sh.at[pl.ds(tile_base, ROWS_PER_TILE)], bounce)
    pltpu.sync_copy(bounce, degc_hbm.at[cid, pl.ds(tile_base, ROWS_PER_TILE)])


@functools.partial(
    pl.kernel,
    out_type=(
        jax.ShapeDtypeStruct((NC, NPAD, D_FEAT), jnp.float32),
    ),
    mesh=_mesh,
    scratch_types=[
        pltpu.VMEM((CPW // 2, CH), jnp.int32),
        pltpu.VMEM((CPW // 2, CH), jnp.int32),
        pltpu.VMEM((2, CH, D_FEAT), jnp.float32),
        pltpu.VMEM_SHARED((NPAD, D_FEAT), jnp.float32),
        pltpu.SemaphoreType.DMA,
        pltpu.SemaphoreType.DMA,
        pltpu.SemaphoreType.DMA,
    ],
)
def _spmm_kernel(scaled_hbm, edge_hbm, out_hbm,
                 ridx2, cidx2, gbuf, acc_sh, gsem, ssem, dsem):
    cid = lax.axis_index("c")
    sid = lax.axis_index("s")
    wid = cid * NS + sid

    def fill16(r, _):
        for k in range(D_FEAT // 16):
            gbuf[0, r, pl.ds(k * 16, 16)] = jnp.zeros((16,), jnp.float32)
        return 0
    lax.fori_loop(0, CH, fill16, 0)

    tile_base = sid * ROWS_PER_TILE
    for k in range(N_DRAIN):
        pltpu.sync_copy(gbuf.at[0],
                        acc_sh.at[pl.ds(tile_base + k * DRAIN_BLK, DRAIN_BLK)])

    plsc.subcore_barrier()

    SCH = CPW // 2
    for h in range(2):
        pltpu.sync_copy(edge_hbm.at[0, 2 * wid + h], ridx2)
        pltpu.sync_copy(edge_hbm.at[1, 2 * wid + h], cidx2)
        pltpu.async_copy(scaled_hbm.at[cidx2.at[0]], gbuf.at[0], gsem)

        def body(j, _):
            b = j % 2

            @pl.when(j + 1 < SCH)
            def _():
                pltpu.async_copy(scaled_hbm.at[cidx2.at[j + 1]], gbuf.at[1 - b],
                                 gsem)

            pltpu.make_async_copy(scaled_hbm.at[cidx2.at[j]], gbuf.at[b],
                                  gsem).wait()
            pltpu.sync_copy(gbuf.at[b], acc_sh.at[ridx2.at[j]], add=True)
            return 0
        lax.fori_loop(0, SCH, body, 0)

    plsc.subcore_barrier()
    for k in range(N_DRAIN):
        b = k % 2
        rows = pl.ds(tile_base + k * DRAIN_BLK, DRAIN_BLK)
        if k >= 2:
            prev = pl.ds(tile_base + (k - 2) * DRAIN_BLK, DRAIN_BLK)
            pltpu.make_async_copy(gbuf.at[b], out_hbm.at[cid, prev], dsem).wait()
        pltpu.sync_copy(acc_sh.at[rows], gbuf.at[b])
        pltpu.async_copy(gbuf.at[b], out_hbm.at[cid, rows], dsem)
    for k in range(N_DRAIN - 2, N_DRAIN):
        rows = pl.ds(tile_base + k * DRAIN_BLK, DRAIN_BLK)
        pltpu.make_async_copy(gbuf.at[k % 2], out_hbm.at[cid, rows], dsem).wait()


def _prep_body(degr_ref, degc_ref, emb_ref, scaled_ref, rrow_ref):
    degr = degr_ref[0] + degr_ref[1]
    degc = degc_ref[0] + degc_ref[1]
    rrow_ref[...] = lax.rsqrt(jnp.maximum(degr, 1.0))
    rcol = lax.rsqrt(jnp.maximum(degc, 1.0))
    rcol_n = rcol.reshape(NPAD)[:N_NODES]
    scaled_ref[:N_NODES, :] = emb_ref[...] * rcol_n[:, None]
    scaled_ref[N_NODES:, :] = jnp.zeros((NPAD - N_NODES, D_FEAT), jnp.float32)


def _finish_body(parts_ref, rrow_ref, out_ref):
    acc = parts_ref[0, :N_NODES, :] + parts_ref[1, :N_NODES, :]
    rrow = rrow_ref[...].reshape(NPAD)[:N_NODES]
    out_ref[...] = acc * rrow[:, None]


def kernel(embeddings, edge_index):
    npad_e = NCHUNKS_PAD * CH - N_EDGES
    pad_row = N_NODES + jnp.arange(npad_e, dtype=jnp.int32) % (NPAD - N_NODES)
    pad = jnp.broadcast_to(pad_row, (2, npad_e))
    edge3 = jnp.concatenate([edge_index, pad], axis=1).reshape(
        2, NW * 2, CPW // 2, CH)
    degr_p, degc_p = _deg_kernel(edge3)
    scaled, rrow = pl.pallas_call(
        _prep_body,
        out_shape=(
            jax.ShapeDtypeStruct((NPAD, D_FEAT), jnp.float32),
            jax.ShapeDtypeStruct((NPAD // 128, 128), jnp.float32),
        ),
    )(degr_p.reshape(NC, NPAD // 128, 128),
      degc_p.reshape(NC, NPAD // 128, 128),
      embeddings)
    (parts,) = _spmm_kernel(scaled, edge3)
    out = pl.pallas_call(
        _finish_body,
        out_shape=jax.ShapeDtypeStruct((N_NODES, D_FEAT), jnp.float32),
    )(parts, rrow)
    return out
